# Initial kernel scaffold; baseline (speedup 1.0000x reference)
import jax
import jax.numpy as jnp
from jax.experimental import pallas as pl


def kernel(x0, x1, x2):
    raise NotImplementedError("write your pallas kernel here")


# speedup vs baseline: 2.1944x; 2.1944x over previous
import numpy as np
import jax
import jax.numpy as jnp
from jax.experimental import pallas as pl

_SIZES = (76, 38, 19)
_ANCHORS = {76: [[28, 28], [46, 45], [64, 66]],
            38: [[102, 74], [78, 115], [132, 113]],
            19: [[149, 163], [174, 268], [257, 176]]}


def _build_params(s: int) -> np.ndarray:
    n = s * s
    stride = float(608 // s)
    par = np.zeros((n, 16), dtype=np.float32)
    p = np.arange(n, dtype=np.float32)
    par[:, 0] = (np.mod(p, s) - 0.025) * stride
    par[:, 1] = (np.floor_divide(p, s) - 0.025) * stride
    par[:, 2] = 1.05 * stride
    for a in range(3):
        par[:, 3 + 2 * a] = float(_ANCHORS[s][a][0])
        par[:, 4 + 2 * a] = float(_ANCHORS[s][a][1])
    return par


_PARAMS = {s: _build_params(s) for s in _SIZES}


def _decode_body(anchors, x_ref, par_ref, out_ref):
    x = x_ref[0]
    n = x.shape[1]
    par = par_ref[...]
    c = jax.lax.broadcasted_iota(jnp.int32, (n, 85), 1)

    for a, (aw, ah) in enumerate(anchors):
        t = x[85 * a:85 * (a + 1), :].T
        e = jnp.exp(t)
        sig = jax.nn.sigmoid(t)
        add = jnp.where(c == 0, par[:, 0:1], par[:, 1:2])
        xyv = sig * par[:, 2:3] + add
        whv = e * jnp.where(c == 2, jnp.float32(aw), jnp.float32(ah))
        res = jnp.where(c < 2, xyv, jnp.where(c < 4, whv, sig))
        out_ref[0:1, pl.Slice(a, n, 3), :] = res[None]


def _decode_scale(x, s):
    b = x.shape[0]
    n = s * s
    f = x.reshape(b, 255, n)
    par = jnp.asarray(_PARAMS[s])
    import functools
    return pl.pallas_call(
        functools.partial(_decode_body, _ANCHORS[s]),
        grid=(b,),
        in_specs=[
            pl.BlockSpec((1, 255, n), lambda i: (i, 0, 0)),
            pl.BlockSpec((n, 16), lambda i: (0, 0)),
        ],
        out_specs=pl.BlockSpec((1, 3 * n, 85), lambda i: (i, 0, 0)),
        out_shape=jax.ShapeDtypeStruct((b, 3 * n, 85), jnp.float32),
    )(f, par)


def kernel(x0, x1, x2):
    outs = [_decode_scale(x, s) for x, s in zip((x0, x1, x2), _SIZES)]
    return jnp.concatenate(outs, axis=1)
